# Initial kernel scaffold; baseline (speedup 1.0000x reference)
import jax
import jax.numpy as jnp
from jax.experimental import pallas as pl


def kernel(x, W_router, w1, w2):
    raise NotImplementedError("write your pallas kernel here")


# speedup vs baseline: 1.2275x; 1.2275x over previous
import functools

import jax
import jax.numpy as jnp
from jax import lax
from jax.experimental import pallas as pl
from jax.experimental.pallas import tpu as pltpu
from jax.experimental.pallas import tpu_sc as plsc

H = 1024
F = 4096
E = 8
TOP_K = 2
TM = 256


def _router_body(x_ref, wr_ref, a1_ref, a2_ref, w1_ref, w2_ref):
    xb = x_ref[...]
    wr = wr_ref[...]
    logits = lax.dot_general(xb, wr, (((1,), (1,)), ((), ())),
                             preferred_element_type=jnp.float32)
    m = jnp.max(logits, axis=1, keepdims=True)
    ex = jnp.exp(logits - m)
    sm = ex / jnp.sum(ex, axis=1, keepdims=True)
    cols = lax.broadcasted_iota(jnp.int32, sm.shape, 1)
    w1v = jnp.max(sm, axis=1, keepdims=True)
    a1v = jnp.min(jnp.where(sm == w1v, cols, E), axis=1, keepdims=True)
    sm2 = jnp.where(cols == a1v, -1.0, sm)
    w2v = jnp.max(sm2, axis=1, keepdims=True)
    a2v = jnp.min(jnp.where(sm2 == w2v, cols, E), axis=1, keepdims=True)
    a1_ref[...] = a1v
    a2_ref[...] = a2v
    w1_ref[...] = w1v
    w2_ref[...] = w2v


def _router(xf, W_router):
    n = xf.shape[0]
    return pl.pallas_call(
        _router_body,
        out_shape=[
            jax.ShapeDtypeStruct((n, 1), jnp.int32),
            jax.ShapeDtypeStruct((n, 1), jnp.int32),
            jax.ShapeDtypeStruct((n, 1), jnp.float32),
            jax.ShapeDtypeStruct((n, 1), jnp.float32),
        ],
    )(xf, W_router)


def _sc_gather_rows(table, idx, chunk=64):
    n_idx = idx.shape[0]
    h = table.shape[1]
    info = plsc.get_sparse_core_info()
    nw = info.num_cores * info.num_subcores
    rows_per = n_idx // nw
    n_chunks = rows_per // chunk
    mesh = plsc.VectorSubcoreMesh(core_axis_name="c", subcore_axis_name="s")

    @functools.partial(
        pl.kernel,
        mesh=mesh,
        out_type=jax.ShapeDtypeStruct((n_idx, h), jnp.float32),
        scratch_types=[
            pltpu.VMEM((chunk,), jnp.int32),
            pltpu.VMEM((chunk, h), jnp.float32),
            pltpu.SemaphoreType.DMA,
        ],
    )
    def k(table_hbm, idx_hbm, out_hbm, idx_v, rows_v, sem):
        wid = lax.axis_index("s") * info.num_cores + lax.axis_index("c")
        base = wid * rows_per
        for c in range(n_chunks):
            off = base + c * chunk
            pltpu.sync_copy(idx_hbm.at[pl.ds(off, chunk)], idx_v)
            pltpu.async_copy(table_hbm.at[idx_v], rows_v, sem).wait()
            pltpu.sync_copy(rows_v, out_hbm.at[pl.ds(off, chunk)])

    return k(table, idx)


def _gemm_body(te_ref, xs_ref, w1_ref, w2_ref, out_ref):
    xb = xs_ref[...].astype(jnp.bfloat16)
    pre = lax.dot_general(xb, w1_ref[0], (((1,), (1,)), ((), ())),
                          preferred_element_type=jnp.float32)
    act = jax.nn.gelu(pre, approximate=True).astype(jnp.bfloat16)
    out_ref[...] = lax.dot_general(act, w2_ref[0], (((1,), (0,)), ((), ())),
                                   preferred_element_type=jnp.float32)


def _grouped_gemm(xs, w1c, w2c, tile_expert, n_tiles):
    grid_spec = pltpu.PrefetchScalarGridSpec(
        num_scalar_prefetch=1,
        grid=(n_tiles,),
        in_specs=[
            pl.BlockSpec((TM, H), lambda m, te: (m, 0)),
            pl.BlockSpec((1, F, H), lambda m, te: (te[m], 0, 0)),
            pl.BlockSpec((1, F, H), lambda m, te: (te[m], 0, 0)),
        ],
        out_specs=pl.BlockSpec((TM, H), lambda m, te: (m, 0)),
    )
    return pl.pallas_call(
        _gemm_body,
        grid_spec=grid_spec,
        out_shape=jax.ShapeDtypeStruct((n_tiles * TM, H), jnp.float32),
        compiler_params=pltpu.CompilerParams(
            dimension_semantics=("arbitrary",)),
    )(tile_expert, xs, w1c, w2c)


def _combine_body(yun_ref, wa_ref, wb_ref, out_ref):
    out_ref[...] = (yun_ref[:, :H] * wa_ref[...]
                    + yun_ref[:, H:] * wb_ref[...])


def _combine(yun2, wa, wb):
    n = yun2.shape[0]
    bt = 512
    return pl.pallas_call(
        _combine_body,
        grid=(n // bt,),
        in_specs=[
            pl.BlockSpec((bt, 2 * H), lambda i: (i, 0)),
            pl.BlockSpec((bt, 1), lambda i: (i, 0)),
            pl.BlockSpec((bt, 1), lambda i: (i, 0)),
        ],
        out_specs=pl.BlockSpec((bt, H), lambda i: (i, 0)),
        out_shape=jax.ShapeDtypeStruct((n, H), jnp.float32),
    )(yun2, wa, wb)


def kernel(x, W_router, w1, w2):
    in_shape = x.shape
    xf = x.reshape(-1, H)
    n = xf.shape[0]
    a_tot = n * TOP_K
    pt = a_tot + E * TM
    n_tiles = pt // TM

    a1, a2, wv1, wv2 = _router(xf, W_router)

    e_flat = jnp.stack([a1[:, 0], a2[:, 0]], axis=1).reshape(-1)
    onehot = (e_flat[:, None] == jnp.arange(E)[None, :]).astype(jnp.int32)
    within = jnp.cumsum(onehot, axis=0) - onehot
    rank = jnp.take_along_axis(within, e_flat[:, None], axis=1)[:, 0]
    counts = jnp.sum(onehot, axis=0)
    padded = ((counts + TM - 1) // TM) * TM
    off_dst = jnp.concatenate([jnp.zeros((1,), jnp.int32),
                               jnp.cumsum(padded)[:-1].astype(jnp.int32)])
    dst_a = off_dst[e_flat] + rank
    slot_token = jnp.zeros((pt,), jnp.int32).at[dst_a].set(
        jnp.arange(a_tot, dtype=jnp.int32) // TOP_K)
    bounds = jnp.cumsum(padded)
    tile_expert = jnp.clip(
        jnp.searchsorted(bounds, jnp.arange(n_tiles, dtype=jnp.int32) * TM,
                         side="right").astype(jnp.int32), 0, E - 1)

    xs = _sc_gather_rows(xf, slot_token)

    w1c = w1.astype(jnp.bfloat16).reshape(E, F, H)
    w2c = w2.astype(jnp.bfloat16).reshape(E, F, H)
    ys = _grouped_gemm(xs, w1c, w2c, tile_expert, n_tiles)

    yun = _sc_gather_rows(ys, dst_a)
    out = _combine(yun.reshape(n, TOP_K * H), wv1, wv2)
    return out.reshape(in_shape)
